# Initial kernel scaffold; baseline (speedup 1.0000x reference)
import jax
import jax.numpy as jnp
from jax.experimental import pallas as pl


def kernel(outputs, targets):
    raise NotImplementedError("write your pallas kernel here")


# speedup vs baseline: 1.7207x; 1.7207x over previous
import functools

import jax
import jax.numpy as jnp
from jax import lax
from jax.experimental import pallas as pl
from jax.experimental.pallas import tpu as pltpu

_SUBLANES = 8


def _cdiv(a, b):
    return (a + b - 1) // b


def _round_up(x, m):
    return ((x + m - 1) // m) * m


def _rows_kernel(o_ref, t_ref, out_ref, *, gamma, n_rows, tr):
    diff = o_ref[...] - t_ref[...]
    sq = diff * diff
    row_loss = jnp.sum(sq, axis=-1, keepdims=True)
    if n_rows % tr != 0:
        limit = n_rows - pl.program_id(0) * tr
        rows = lax.broadcasted_iota(jnp.int32, row_loss.shape, 0)
        row_loss = jnp.where(rows < limit, row_loss, 0.0)
    w = 1.0 - jnp.exp(-row_loss)
    g = int(gamma)
    wg = w
    for _ in range(g - 1):
        wg = wg * w
    focal = wg * row_loss
    out_ref[...] = jnp.sum(focal, axis=0, keepdims=True).reshape(1, 1, 1)


def kernel(outputs, targets):
    gamma = 2
    B, S, D = outputs.shape
    n_items = B * S

    o2 = outputs.reshape(n_items, D)
    t2 = targets.reshape(n_items, D)

    tr = 8192
    n_pad = _round_up(n_items, _SUBLANES)
    tr = min(tr, n_pad)
    grid = _cdiv(n_items, tr)

    kern = functools.partial(_rows_kernel, gamma=gamma, n_rows=n_items,
                             tr=tr)
    partials = pl.pallas_call(
        kern,
        out_shape=jax.ShapeDtypeStruct((grid, 1, 1), jnp.float32),
        grid_spec=pltpu.PrefetchScalarGridSpec(
            num_scalar_prefetch=0,
            grid=(grid,),
            in_specs=[
                pl.BlockSpec((tr, D), lambda i: (i, 0)),
                pl.BlockSpec((tr, D), lambda i: (i, 0)),
            ],
            out_specs=pl.BlockSpec((1, 1, 1), lambda i: (i, 0, 0)),
        ),
        compiler_params=pltpu.CompilerParams(
            dimension_semantics=("parallel",),
            vmem_limit_bytes=64 * 1024 * 1024,
        ),
    )(o2, t2)
    return jnp.sum(partials) / float(n_items)
